# baseline (speedup 1.0000x reference)
import dataclasses
import functools

import jax
import jax.numpy as jnp
from jax import lax
from jax.experimental import pallas as pl
from jax.experimental.pallas import tpu as pltpu
from jax.experimental.pallas import tpu_sc as plsc

_NC, _NS, _L = 2, 16, 16
_NW = _NC * _NS
_B = 16384
_R = 196
_K = 512
_BW = _B // _NW
_CH = 64
_NCHUNK = _BW // _CH

_RP = 256
_RFULL = _R // _L
_RTAIL = _R - _RFULL * _L
_RTOFF = _R - _L
_KFULL = _K // _L


def _sc_body(idx_hbm, wr_hbm, wk_hbm, outr_hbm, outk_hbm,
             idx_v, buf_r, buf_k, obuf_r, obuf_k, sem_r, sem_k):
  wid = lax.axis_index("s") * _NC + lax.axis_index("c")
  base = wid * _BW
  pltpu.sync_copy(idx_hbm.at[pl.ds(base, _BW)], idx_v)

  lane = lax.iota(jnp.int32, 16)

  @pl.loop(0, _NCHUNK)
  def _chunk(c):
    cb = c * _CH
    cp_r = pltpu.async_copy(wr_hbm.at[idx_v.at[pl.ds(cb, _CH)]], buf_r, sem_r)
    cp_k = pltpu.async_copy(wk_hbm.at[idx_v.at[pl.ds(cb, _CH)]], buf_k, sem_k)
    cp_r.wait()
    cp_k.wait()

    @pl.loop(0, _CH)
    def _row(r):
      def racc(j, acc):
        return acc + jnp.abs(buf_r[r, pl.ds(j * _L, _L)])
      acc_r = lax.fori_loop(0, _RFULL, racc, jnp.zeros((_L,), jnp.float32))
      tail = jnp.abs(buf_r[r, pl.ds(_RFULL * _L, _L)])
      acc_r = acc_r + jnp.where(lane < _RTAIL, tail, 0.0)
      norm_r = jnp.broadcast_to(jnp.sum(acc_r), (_L,))
      scale_r = jnp.where(norm_r > 1.0, 1.0 / (norm_r + 1e-7),
                          jnp.ones((_L,), jnp.float32))

      @pl.loop(0, _RFULL)
      def _rw(j):
        obuf_r[r, pl.ds(j * _L, _L)] = buf_r[r, pl.ds(j * _L, _L)] * scale_r
      obuf_r[r, pl.ds(_RTOFF, _L)] = buf_r[r, pl.ds(_RTOFF, _L)] * scale_r

      def kacc(j, acc):
        return acc + jnp.abs(buf_k[r, pl.ds(j * _L, _L)])
      acc_k = lax.fori_loop(0, _KFULL, kacc, jnp.zeros((_L,), jnp.float32))
      norm_k = jnp.broadcast_to(jnp.sum(acc_k), (_L,))
      scale_k = jnp.where(norm_k > 1.0, 1.0 / (norm_k + 1e-7),
                          jnp.ones((_L,), jnp.float32))

      @pl.loop(0, _KFULL)
      def _kw(j):
        obuf_k[r, pl.ds(j * _L, _L)] = buf_k[r, pl.ds(j * _L, _L)] * scale_k

    pltpu.sync_copy(obuf_r, outr_hbm.at[pl.ds(base + cb, _CH)])
    pltpu.sync_copy(obuf_k, outk_hbm.at[pl.ds(base + cb, _CH)])


@jax.jit
def _run(inds, W_region, W_kernel):
  mesh = plsc.VectorSubcoreMesh(core_axis_name="c", subcore_axis_name="s",
                                num_cores=_NC, num_subcores=_NS)
  cp = pltpu.CompilerParams()
  if "needs_layout_passes" in pltpu.CompilerParams.__dataclass_fields__:
    cp = dataclasses.replace(cp, needs_layout_passes=False)
  k = pl.kernel(
      _sc_body,
      out_type=(
          jax.ShapeDtypeStruct((_B, _R), jnp.float32),
          jax.ShapeDtypeStruct((_B, _K), jnp.float32),
      ),
      mesh=mesh,
      scratch_types=[
          pltpu.VMEM((_BW,), jnp.int32),
          pltpu.VMEM((_CH, _RP), jnp.float32),
          pltpu.VMEM((_CH, _K), jnp.float32),
          pltpu.VMEM((_CH, _R), jnp.float32),
          pltpu.VMEM((_CH, _K), jnp.float32),
          pltpu.SemaphoreType.DMA,
          pltpu.SemaphoreType.DMA,
      ],
      compiler_params=cp,
  )
  w_region_p = jnp.pad(W_region, ((0, 0), (0, _RP - _R)))
  return k(inds, w_region_p, W_kernel)


def kernel(inds, W_region, W_kernel):
  return _run(inds.astype(jnp.int32), W_region, W_kernel)


# speedup vs baseline: 1.8480x; 1.8480x over previous
import dataclasses
import functools

import jax
import jax.numpy as jnp
from jax import lax
from jax.experimental import pallas as pl
from jax.experimental.pallas import tpu as pltpu
from jax.experimental.pallas import tpu_sc as plsc

_NC, _NS, _L = 2, 16, 16
_NW = _NC * _NS
_B = 16384
_R = 196
_K = 512
_BW = _B // _NW
_CH = 64
_NCHUNK = _BW // _CH

_RP = 256
_RFULL = _R // _L
_RTAIL = _R - _RFULL * _L
_RTOFF = _R - _L
_KFULL = _K // _L


_PADBLK = 2000


def _pad_body(src_ref, dst_ref):
  dst_ref[:, :_R] = src_ref[...]
  dst_ref[:, _R:] = jnp.zeros((_PADBLK, _RP - _R), jnp.float32)


def _pad_region(w_region):
  n = w_region.shape[0]
  return pl.pallas_call(
      _pad_body,
      grid=(n // _PADBLK,),
      in_specs=[pl.BlockSpec((_PADBLK, _R), lambda i: (i, 0))],
      out_specs=pl.BlockSpec((_PADBLK, _RP), lambda i: (i, 0)),
      out_shape=jax.ShapeDtypeStruct((n, _RP), jnp.float32),
  )(w_region)


def _sc_body(idx_hbm, wr_hbm, wk_hbm, outr_hbm, outk_hbm,
             idx_v, buf_r, buf_k, obuf_r, obuf_k, sem_r, sem_k):
  wid = lax.axis_index("s") * _NC + lax.axis_index("c")
  base = wid * _BW
  pltpu.sync_copy(idx_hbm.at[pl.ds(base, _BW)], idx_v)

  lane = lax.iota(jnp.int32, 16)

  @pl.loop(0, _NCHUNK)
  def _chunk(c):
    cb = c * _CH
    cp_r = pltpu.async_copy(wr_hbm.at[idx_v.at[pl.ds(cb, _CH)]], buf_r, sem_r)
    cp_k = pltpu.async_copy(wk_hbm.at[idx_v.at[pl.ds(cb, _CH)]], buf_k, sem_k)
    cp_r.wait()
    cp_k.wait()

    @pl.loop(0, _CH)
    def _row(r):
      def racc(j, acc):
        return acc + jnp.abs(buf_r[r, pl.ds(j * _L, _L)])
      acc_r = lax.fori_loop(0, _RFULL, racc, jnp.zeros((_L,), jnp.float32))
      tail = jnp.abs(buf_r[r, pl.ds(_RFULL * _L, _L)])
      acc_r = acc_r + jnp.where(lane < _RTAIL, tail, 0.0)
      norm_r = jnp.broadcast_to(jnp.sum(acc_r), (_L,))
      scale_r = jnp.where(norm_r > 1.0, 1.0 / (norm_r + 1e-7),
                          jnp.ones((_L,), jnp.float32))

      @pl.loop(0, _RFULL)
      def _rw(j):
        obuf_r[r, pl.ds(j * _L, _L)] = buf_r[r, pl.ds(j * _L, _L)] * scale_r
      obuf_r[r, pl.ds(_RTOFF, _L)] = buf_r[r, pl.ds(_RTOFF, _L)] * scale_r

      def kacc(j, acc):
        return acc + jnp.abs(buf_k[r, pl.ds(j * _L, _L)])
      acc_k = lax.fori_loop(0, _KFULL, kacc, jnp.zeros((_L,), jnp.float32))
      norm_k = jnp.broadcast_to(jnp.sum(acc_k), (_L,))
      scale_k = jnp.where(norm_k > 1.0, 1.0 / (norm_k + 1e-7),
                          jnp.ones((_L,), jnp.float32))

      @pl.loop(0, _KFULL)
      def _kw(j):
        obuf_k[r, pl.ds(j * _L, _L)] = buf_k[r, pl.ds(j * _L, _L)] * scale_k

    pltpu.sync_copy(obuf_r, outr_hbm.at[pl.ds(base + cb, _CH)])
    pltpu.sync_copy(obuf_k, outk_hbm.at[pl.ds(base + cb, _CH)])


@jax.jit
def _run(inds, W_region, W_kernel):
  mesh = plsc.VectorSubcoreMesh(core_axis_name="c", subcore_axis_name="s",
                                num_cores=_NC, num_subcores=_NS)
  cp = pltpu.CompilerParams()
  if "needs_layout_passes" in pltpu.CompilerParams.__dataclass_fields__:
    cp = dataclasses.replace(cp, needs_layout_passes=False)
  k = pl.kernel(
      _sc_body,
      out_type=(
          jax.ShapeDtypeStruct((_B, _R), jnp.float32),
          jax.ShapeDtypeStruct((_B, _K), jnp.float32),
      ),
      mesh=mesh,
      scratch_types=[
          pltpu.VMEM((_BW,), jnp.int32),
          pltpu.VMEM((_CH, _RP), jnp.float32),
          pltpu.VMEM((_CH, _K), jnp.float32),
          pltpu.VMEM((_CH, _R), jnp.float32),
          pltpu.VMEM((_CH, _K), jnp.float32),
          pltpu.SemaphoreType.DMA,
          pltpu.SemaphoreType.DMA,
      ],
      compiler_params=cp,
  )
  return k(inds, _pad_region(W_region), W_kernel)


def kernel(inds, W_region, W_kernel):
  return _run(inds.astype(jnp.int32), W_region, W_kernel)


# speedup vs baseline: 3.0362x; 1.6430x over previous
import dataclasses
import functools

import jax
import jax.numpy as jnp
from jax import lax
from jax.experimental import pallas as pl
from jax.experimental.pallas import tpu as pltpu
from jax.experimental.pallas import tpu_sc as plsc

_NC, _NS, _L = 2, 16, 16
_NW = _NC * _NS
_B = 16384
_R = 196
_K = 512
_BW = _B // _NW
_CH = 32
_NCHUNK = _BW // _CH

_RP = 256
_RFULL = _R // _L
_RTAIL = _R - _RFULL * _L
_RTOFF = _R - _L
_KFULL = _K // _L


_PADBLK = 2000


def _pad_body(src_ref, dst_ref):
  dst_ref[:, :_R] = src_ref[...]
  dst_ref[:, _R:] = jnp.zeros((_PADBLK, _RP - _R), jnp.float32)


def _pad_region(w_region):
  n = w_region.shape[0]
  return pl.pallas_call(
      _pad_body,
      grid=(n // _PADBLK,),
      in_specs=[pl.BlockSpec((_PADBLK, _R), lambda i: (i, 0))],
      out_specs=pl.BlockSpec((_PADBLK, _RP), lambda i: (i, 0)),
      out_shape=jax.ShapeDtypeStruct((n, _RP), jnp.float32),
  )(w_region)


def _tree_sum(vals):
  vals = list(vals)
  while len(vals) > 1:
    nxt = [a + b for a, b in zip(vals[0::2], vals[1::2])]
    if len(vals) % 2:
      nxt.append(vals[-1])
    vals = nxt
  return vals[0]


def _scale_from(acc):
  norm = jnp.broadcast_to(jnp.sum(acc), (_L,))
  return jnp.where(norm > 1.0, 1.0 / (norm + 1e-7),
                   jnp.ones((_L,), jnp.float32))


def _sc_body(idx_hbm, wr_hbm, wk_hbm, outr_hbm, outk_hbm,
             idx_v, buf_r0, buf_r1, buf_k0, buf_k1,
             obuf_r0, obuf_r1, obuf_k0, obuf_k1,
             gsem_r0, gsem_r1, gsem_k0, gsem_k1,
             osem_r0, osem_r1, osem_k0, osem_k1):
  wid = lax.axis_index("s") * _NC + lax.axis_index("c")
  base = wid * _BW
  pltpu.sync_copy(idx_hbm.at[pl.ds(base, _BW)], idx_v)

  lane = lax.iota(jnp.int32, 16)
  bufs_r = (buf_r0, buf_r1)
  bufs_k = (buf_k0, buf_k1)
  obufs_r = (obuf_r0, obuf_r1)
  obufs_k = (obuf_k0, obuf_k1)
  gsems_r = (gsem_r0, gsem_r1)
  gsems_k = (gsem_k0, gsem_k1)
  osems_r = (osem_r0, osem_r1)
  osems_k = (osem_k0, osem_k1)

  def gather_descs(cc):
    p = cc % 2
    s = idx_v.at[pl.ds(cc * _CH, _CH)]
    return (pltpu.make_async_copy(wr_hbm.at[s], bufs_r[p], gsems_r[p]),
            pltpu.make_async_copy(wk_hbm.at[s], bufs_k[p], gsems_k[p]))

  def out_descs(cc):
    p = cc % 2
    d = pl.ds(base + cc * _CH, _CH)
    return (pltpu.make_async_copy(obufs_r[p], outr_hbm.at[d], osems_r[p]),
            pltpu.make_async_copy(obufs_k[p], outk_hbm.at[d], osems_k[p]))

  def compute(p):
    buf_r, buf_k = bufs_r[p], bufs_k[p]
    obuf_r, obuf_k = obufs_r[p], obufs_k[p]

    @pl.loop(0, _CH)
    def _row_r(r):
      wins = [buf_r[r, pl.ds(j * _L, _L)] for j in range(_RFULL)]
      tail = buf_r[r, pl.ds(_RTOFF, _L)]
      acc = _tree_sum([jnp.abs(w) for w in wins]
                      + [jnp.where(lane >= _L - _RTAIL, jnp.abs(tail), 0.0)])
      scale = _scale_from(acc)
      for j in range(_RFULL):
        obuf_r[r, pl.ds(j * _L, _L)] = wins[j] * scale
      obuf_r[r, pl.ds(_RTOFF, _L)] = tail * scale

    @pl.loop(0, _CH)
    def _row_k(r):
      wins = [buf_k[r, pl.ds(j * _L, _L)] for j in range(_KFULL)]
      scale = _scale_from(_tree_sum([jnp.abs(w) for w in wins]))
      for j in range(_KFULL):
        obuf_k[r, pl.ds(j * _L, _L)] = wins[j] * scale

  for d in gather_descs(0):
    d.start()
  for d in gather_descs(1):
    d.start()
  for cc in range(_NCHUNK):
    for d in gather_descs(cc):
      d.wait()
    if cc >= 2:
      for d in out_descs(cc - 2):
        d.wait()
    compute(cc % 2)
    for d in out_descs(cc):
      d.start()
    if cc + 2 < _NCHUNK:
      for d in gather_descs(cc + 2):
        d.start()
  for d in out_descs(_NCHUNK - 2):
    d.wait()
  for d in out_descs(_NCHUNK - 1):
    d.wait()


@jax.jit
def _run(inds, W_region, W_kernel):
  mesh = plsc.VectorSubcoreMesh(core_axis_name="c", subcore_axis_name="s",
                                num_cores=_NC, num_subcores=_NS)
  cp = pltpu.CompilerParams()
  if "needs_layout_passes" in pltpu.CompilerParams.__dataclass_fields__:
    cp = dataclasses.replace(cp, needs_layout_passes=False)
  k = pl.kernel(
      _sc_body,
      out_type=(
          jax.ShapeDtypeStruct((_B, _R), jnp.float32),
          jax.ShapeDtypeStruct((_B, _K), jnp.float32),
      ),
      mesh=mesh,
      scratch_types=(
          [pltpu.VMEM((_BW,), jnp.int32)]
          + [pltpu.VMEM((_CH, _RP), jnp.float32)] * 2
          + [pltpu.VMEM((_CH, _K), jnp.float32)] * 2
          + [pltpu.VMEM((_CH, _R), jnp.float32)] * 2
          + [pltpu.VMEM((_CH, _K), jnp.float32)] * 2
          + [pltpu.SemaphoreType.DMA] * 8
      ),
      compiler_params=cp,
  )
  return k(inds, _pad_region(W_region), W_kernel)


def kernel(inds, W_region, W_kernel):
  return _run(inds.astype(jnp.int32), W_region, W_kernel)


# speedup vs baseline: 3.7292x; 1.2282x over previous
import dataclasses
import functools

import jax
import jax.numpy as jnp
from jax import lax
from jax.experimental import pallas as pl
from jax.experimental.pallas import tpu as pltpu
from jax.experimental.pallas import tpu_sc as plsc

_NC, _NS, _L = 2, 16, 16
_NW = _NC * _NS
_B = 16384
_R = 196
_K = 512
_BW = _B // _NW
_CH = 32
_NCHUNK = _BW // _CH
_KFULL = _K // _L
_IDXH = _B // 2
_DMAX = (_R + _NW - 1) // _NW
_COLS = _B // _NW
_HC = _COLS // 2


def _mesh():
  return plsc.VectorSubcoreMesh(core_axis_name="c", subcore_axis_name="s",
                                num_cores=_NC, num_subcores=_NS)


def _cparams():
  cp = pltpu.CompilerParams()
  if "needs_layout_passes" in pltpu.CompilerParams.__dataclass_fields__:
    cp = dataclasses.replace(cp, needs_layout_passes=False)
  return cp


def _tree_sum(vals):
  vals = list(vals)
  while len(vals) > 1:
    nxt = [a + b for a, b in zip(vals[0::2], vals[1::2])]
    if len(vals) % 2:
      nxt.append(vals[-1])
    vals = nxt
  return vals[0]


def _scale_vec(norm):
  return jnp.where(norm > 1.0, 1.0 / (norm + 1e-7),
                   jnp.ones((_L,), jnp.float32))


def _k1_body(idx_hbm, wrt_hbm, outt_hbm, idx_v, row_v, gout_v, sem):
  wid = lax.axis_index("s") * _NC + lax.axis_index("c")
  pltpu.sync_copy(idx_hbm, idx_v)

  @pl.loop(0, _DMAX)
  def _rows(j):
    d = wid + j * _NW

    @pl.when(d < _R)
    def _():
      pltpu.async_copy(wrt_hbm.at[d], row_v, sem).wait()
      for h in range(2):
        @pl.loop(0, _IDXH // _L, unroll=8)
        def _gather(w):
          iv = idx_v[pl.ds(h * _IDXH + w * _L, _L)]
          gout_v[pl.ds(w * _L, _L)] = plsc.load_gather(row_v, [iv])
        pltpu.sync_copy(gout_v, outt_hbm.at[d, pl.ds(h * _IDXH, _IDXH)])


def _k2_body(idx_hbm, wk_hbm, outtr_hbm, outt_hbm, outk_hbm,
             idx_v, slab_v, buf_k0, buf_k1, obuf_k0, obuf_k1,
             sem_s, gsem_k0, gsem_k1, osem_k0, osem_k1):
  wid = lax.axis_index("s") * _NC + lax.axis_index("c")
  base = wid * _BW
  pltpu.sync_copy(idx_hbm.at[pl.ds(base, _BW)], idx_v)

  bufs_k = (buf_k0, buf_k1)
  obufs_k = (obuf_k0, obuf_k1)
  gsems_k = (gsem_k0, gsem_k1)
  osems_k = (osem_k0, osem_k1)

  slab0 = pltpu.make_async_copy(
      outtr_hbm.at[:, pl.ds(base, _HC)], slab_v, sem_s)
  slab0.start()

  def gather_desc(cc):
    p = cc % 2
    s = idx_v.at[pl.ds(cc * _CH, _CH)]
    return pltpu.make_async_copy(wk_hbm.at[s], bufs_k[p], gsems_k[p])

  def out_desc(cc):
    p = cc % 2
    d = pl.ds(base + cc * _CH, _CH)
    return pltpu.make_async_copy(obufs_k[p], outk_hbm.at[d], osems_k[p])

  def compute(p):
    buf_k, obuf_k = bufs_k[p], obufs_k[p]

    @pl.loop(0, _CH)
    def _row_k(r):
      wins = [buf_k[r, pl.ds(j * _L, _L)] for j in range(_KFULL)]
      norm = jnp.broadcast_to(
          jnp.sum(_tree_sum([jnp.abs(w) for w in wins])), (_L,))
      scale = _scale_vec(norm)
      for j in range(_KFULL):
        obuf_k[r, pl.ds(j * _L, _L)] = wins[j] * scale

  gather_desc(0).start()
  gather_desc(1).start()
  for cc in range(_NCHUNK):
    gather_desc(cc).wait()
    if cc >= 2:
      out_desc(cc - 2).wait()
    compute(cc % 2)
    out_desc(cc).start()
    if cc + 2 < _NCHUNK:
      gather_desc(cc + 2).start()
  out_desc(_NCHUNK - 2).wait()
  out_desc(_NCHUNK - 1).wait()

  nch = _HC // _L

  for h in range(2):
    b0 = base + h * _HC
    if h == 0:
      slab0.wait()
    else:
      pltpu.async_copy(outtr_hbm.at[:, pl.ds(b0, _HC)], slab_v, sem_s).wait()

    def nacc(dd, accs):
      return tuple(accs[c] + jnp.abs(slab_v[dd, pl.ds(c * _L, _L)])
                   for c in range(nch))
    norms = lax.fori_loop(
        0, _R, nacc, tuple(jnp.zeros((_L,), jnp.float32) for _ in range(nch)))
    scales = [_scale_vec(n) for n in norms]

    @pl.loop(0, _R)
    def _scale_rows(dd):
      for c in range(nch):
        slab_v[dd, pl.ds(c * _L, _L)] = (
            slab_v[dd, pl.ds(c * _L, _L)] * scales[c])

    pltpu.async_copy(slab_v, outt_hbm.at[:, pl.ds(b0, _HC)], sem_s).wait()


@jax.jit
def _run(inds, W_region, W_kernel):
  n = W_region.shape[0]
  wrt = W_region.T

  k1 = pl.kernel(
      _k1_body,
      out_type=jax.ShapeDtypeStruct((_R, _B), jnp.float32),
      mesh=_mesh(),
      scratch_types=[
          pltpu.VMEM((_B,), jnp.int32),
          pltpu.VMEM((n,), jnp.float32),
          pltpu.VMEM((_IDXH,), jnp.float32),
          pltpu.SemaphoreType.DMA,
      ],
      compiler_params=_cparams(),
  )
  out_raw = k1(inds, wrt)

  k2 = pl.kernel(
      _k2_body,
      out_type=(
          jax.ShapeDtypeStruct((_R, _B), jnp.float32),
          jax.ShapeDtypeStruct((_B, _K), jnp.float32),
      ),
      mesh=_mesh(),
      scratch_types=(
          [pltpu.VMEM((_BW,), jnp.int32),
           pltpu.VMEM((_R, _HC), jnp.float32)]
          + [pltpu.VMEM((_CH, _K), jnp.float32)] * 4
          + [pltpu.SemaphoreType.DMA] * 5
      ),
      compiler_params=_cparams(),
  )
  out_t, out_k = k2(inds, W_kernel, out_raw)
  return out_t.T, out_k


def kernel(inds, W_region, W_kernel):
  return _run(inds.astype(jnp.int32), W_region, W_kernel)


# speedup vs baseline: 3.7781x; 1.0131x over previous
import dataclasses
import functools

import jax
import jax.numpy as jnp
from jax import lax
from jax.experimental import pallas as pl
from jax.experimental.pallas import tpu as pltpu
from jax.experimental.pallas import tpu_sc as plsc

_NC, _NS, _L = 2, 16, 16
_NW = _NC * _NS
_B = 16384
_R = 196
_K = 512
_BW = _B // _NW
_CH = 32
_NCHUNK = _BW // _CH
_KFULL = _K // _L
_IDXH = _B // 2
_DMAX = (_R + _NW - 1) // _NW
_COLS = _B // _NW
_HC = _COLS // 2


def _mesh():
  return plsc.VectorSubcoreMesh(core_axis_name="c", subcore_axis_name="s",
                                num_cores=_NC, num_subcores=_NS)


def _cparams():
  cp = pltpu.CompilerParams()
  if "needs_layout_passes" in pltpu.CompilerParams.__dataclass_fields__:
    cp = dataclasses.replace(cp, needs_layout_passes=False)
  return cp


def _tree_sum(vals):
  vals = list(vals)
  while len(vals) > 1:
    nxt = [a + b for a, b in zip(vals[0::2], vals[1::2])]
    if len(vals) % 2:
      nxt.append(vals[-1])
    vals = nxt
  return vals[0]


def _scale_vec(norm):
  return jnp.where(norm > 1.0, 1.0 / (norm + 1e-7),
                   jnp.ones((_L,), jnp.float32))


def _k1_body(idx_hbm, wrt_hbm, outt_hbm, idx_v, row_v, gout_v, sem):
  wid = lax.axis_index("s") * _NC + lax.axis_index("c")
  pltpu.sync_copy(idx_hbm, idx_v)

  @pl.loop(0, _DMAX)
  def _rows(j):
    d = wid + j * _NW

    @pl.when(d < _R)
    def _():
      with jax.named_scope("k1_dma_row"):
        pltpu.async_copy(wrt_hbm.at[d], row_v, sem).wait()
      for h in range(2):
        with jax.named_scope("k1_gather"):
          @pl.loop(0, _IDXH // _L, unroll=8)
          def _gather(w):
            iv = idx_v[pl.ds(h * _IDXH + w * _L, _L)]
            gout_v[pl.ds(w * _L, _L)] = plsc.load_gather(row_v, [iv])
        with jax.named_scope("k1_out"):
          pltpu.sync_copy(gout_v, outt_hbm.at[d, pl.ds(h * _IDXH, _IDXH)])


def _k2_body(idx_hbm, wk_hbm, outtr_hbm, outt_hbm, outk_hbm,
             idx_v, slab_v, buf_k0, buf_k1, obuf_k0, obuf_k1,
             sem_s, gsem_k0, gsem_k1, osem_k0, osem_k1):
  wid = lax.axis_index("s") * _NC + lax.axis_index("c")
  base = wid * _BW
  pltpu.sync_copy(idx_hbm.at[pl.ds(base, _BW)], idx_v)

  bufs_k = (buf_k0, buf_k1)
  obufs_k = (obuf_k0, obuf_k1)
  gsems_k = (gsem_k0, gsem_k1)
  osems_k = (osem_k0, osem_k1)

  slab0 = pltpu.make_async_copy(
      outtr_hbm.at[:, pl.ds(base, _HC)], slab_v, sem_s)
  slab0.start()

  def gather_desc(cc):
    p = cc % 2
    s = idx_v.at[pl.ds(cc * _CH, _CH)]
    return pltpu.make_async_copy(wk_hbm.at[s], bufs_k[p], gsems_k[p])

  def out_desc(cc):
    p = cc % 2
    d = pl.ds(base + cc * _CH, _CH)
    return pltpu.make_async_copy(obufs_k[p], outk_hbm.at[d], osems_k[p])

  def compute(p):
    buf_k, obuf_k = bufs_k[p], obufs_k[p]

    @pl.loop(0, _CH)
    def _row_k(r):
      wins = [buf_k[r, pl.ds(j * _L, _L)] for j in range(_KFULL)]
      norm = jnp.broadcast_to(
          jnp.sum(_tree_sum([jnp.abs(w) for w in wins])), (_L,))
      scale = _scale_vec(norm)
      for j in range(_KFULL):
        obuf_k[r, pl.ds(j * _L, _L)] = wins[j] * scale

  gather_desc(0).start()
  gather_desc(1).start()
  for cc in range(_NCHUNK):
    gather_desc(cc).wait()
    if cc >= 2:
      out_desc(cc - 2).wait()
    compute(cc % 2)
    out_desc(cc).start()
    if cc + 2 < _NCHUNK:
      gather_desc(cc + 2).start()
  out_desc(_NCHUNK - 2).wait()
  out_desc(_NCHUNK - 1).wait()

  nch = _HC // _L

  for h in range(2):
    b0 = base + h * _HC
    if h == 0:
      slab0.wait()
    else:
      pltpu.async_copy(outtr_hbm.at[:, pl.ds(b0, _HC)], slab_v, sem_s).wait()

    def nacc(dd, accs):
      return tuple(accs[c] + jnp.abs(slab_v[dd, pl.ds(c * _L, _L)])
                   for c in range(nch))
    norms = lax.fori_loop(
        0, _R, nacc, tuple(jnp.zeros((_L,), jnp.float32) for _ in range(nch)))
    scales = [_scale_vec(n) for n in norms]

    @pl.loop(0, _R)
    def _scale_rows(dd):
      for c in range(nch):
        slab_v[dd, pl.ds(c * _L, _L)] = (
            slab_v[dd, pl.ds(c * _L, _L)] * scales[c])

    pltpu.async_copy(slab_v, outt_hbm.at[:, pl.ds(b0, _HC)], sem_s).wait()


@jax.jit
def _run(inds, W_region, W_kernel):
  n = W_region.shape[0]
  wrt = W_region.T

  k1 = pl.kernel(
      _k1_body,
      out_type=jax.ShapeDtypeStruct((_R, _B), jnp.float32),
      mesh=_mesh(),
      scratch_types=[
          pltpu.VMEM((_B,), jnp.int32),
          pltpu.VMEM((n,), jnp.float32),
          pltpu.VMEM((_IDXH,), jnp.float32),
          pltpu.SemaphoreType.DMA,
      ],
      compiler_params=_cparams(),
  )
  out_raw = k1(inds, wrt)

  k2 = pl.kernel(
      _k2_body,
      out_type=(
          jax.ShapeDtypeStruct((_R, _B), jnp.float32),
          jax.ShapeDtypeStruct((_B, _K), jnp.float32),
      ),
      mesh=_mesh(),
      scratch_types=(
          [pltpu.VMEM((_BW,), jnp.int32),
           pltpu.VMEM((_R, _HC), jnp.float32)]
          + [pltpu.VMEM((_CH, _K), jnp.float32)] * 4
          + [pltpu.SemaphoreType.DMA] * 5
      ),
      compiler_params=_cparams(),
  )
  out_t, out_k = k2(inds, W_kernel, out_raw)
  return out_t.T, out_k


def kernel(inds, W_region, W_kernel):
  return _run(inds.astype(jnp.int32), W_region, W_kernel)


# speedup vs baseline: 3.7853x; 1.0019x over previous
import dataclasses
import functools

import jax
import jax.numpy as jnp
from jax import lax
from jax.experimental import pallas as pl
from jax.experimental.pallas import tpu as pltpu
from jax.experimental.pallas import tpu_sc as plsc

_NC, _NS, _L = 2, 16, 16
_NW = _NC * _NS
_B = 16384
_R = 196
_K = 512
_BW = _B // _NW
_CH = 32
_NCHUNK = _BW // _CH
_KFULL = _K // _L
_IDXH = _B // 2
_DMAX = (_R + _NW - 1) // _NW
_COLS = _B // _NW
_HC = _COLS // 2


def _mesh():
  return plsc.VectorSubcoreMesh(core_axis_name="c", subcore_axis_name="s",
                                num_cores=_NC, num_subcores=_NS)


def _cparams():
  cp = pltpu.CompilerParams()
  if "needs_layout_passes" in pltpu.CompilerParams.__dataclass_fields__:
    cp = dataclasses.replace(cp, needs_layout_passes=False)
  return cp


def _tree_sum(vals):
  vals = list(vals)
  while len(vals) > 1:
    nxt = [a + b for a, b in zip(vals[0::2], vals[1::2])]
    if len(vals) % 2:
      nxt.append(vals[-1])
    vals = nxt
  return vals[0]


def _scale_vec(norm):
  return jnp.where(norm > 1.0, 1.0 / (norm + 1e-7),
                   jnp.ones((_L,), jnp.float32))


def _k1_body(idx_hbm, wrt_hbm, outt_hbm, idx_v, row_v, gout_v, sem):
  wid = lax.axis_index("s") * _NC + lax.axis_index("c")
  pltpu.sync_copy(idx_hbm, idx_v)

  @pl.loop(0, _DMAX)
  def _rows(j):
    d = wid + j * _NW

    @pl.when(d < _R)
    def _():
      with jax.named_scope("k1_dma_row"):
        pltpu.async_copy(wrt_hbm.at[d], row_v, sem).wait()
      for h in range(2):
        with jax.named_scope("k1_gather"):
          @pl.loop(0, _IDXH // _L, unroll=16)
          def _gather(w):
            iv = idx_v[pl.ds(h * _IDXH + w * _L, _L)]
            gout_v[pl.ds(w * _L, _L)] = plsc.load_gather(row_v, [iv])
        with jax.named_scope("k1_out"):
          pltpu.sync_copy(gout_v, outt_hbm.at[d, pl.ds(h * _IDXH, _IDXH)])


def _k2_body(idx_hbm, wk_hbm, outtr_hbm, outt_hbm, outk_hbm,
             idx_v, slab_v, buf_k0, buf_k1, obuf_k0, obuf_k1,
             sem_s, gsem_k0, gsem_k1, osem_k0, osem_k1):
  wid = lax.axis_index("s") * _NC + lax.axis_index("c")
  base = wid * _BW
  pltpu.sync_copy(idx_hbm.at[pl.ds(base, _BW)], idx_v)

  bufs_k = (buf_k0, buf_k1)
  obufs_k = (obuf_k0, obuf_k1)
  gsems_k = (gsem_k0, gsem_k1)
  osems_k = (osem_k0, osem_k1)

  slab0 = pltpu.make_async_copy(
      outtr_hbm.at[:, pl.ds(base, _HC)], slab_v, sem_s)
  slab0.start()

  def gather_desc(cc):
    p = cc % 2
    s = idx_v.at[pl.ds(cc * _CH, _CH)]
    return pltpu.make_async_copy(wk_hbm.at[s], bufs_k[p], gsems_k[p])

  def out_desc(cc):
    p = cc % 2
    d = pl.ds(base + cc * _CH, _CH)
    return pltpu.make_async_copy(obufs_k[p], outk_hbm.at[d], osems_k[p])

  def compute(p):
    buf_k, obuf_k = bufs_k[p], obufs_k[p]

    @pl.loop(0, _CH)
    def _row_k(r):
      wins = [buf_k[r, pl.ds(j * _L, _L)] for j in range(_KFULL)]
      norm = jnp.broadcast_to(
          jnp.sum(_tree_sum([jnp.abs(w) for w in wins])), (_L,))
      scale = _scale_vec(norm)
      for j in range(_KFULL):
        obuf_k[r, pl.ds(j * _L, _L)] = wins[j] * scale

  gather_desc(0).start()
  gather_desc(1).start()
  for cc in range(_NCHUNK):
    gather_desc(cc).wait()
    if cc >= 2:
      out_desc(cc - 2).wait()
    compute(cc % 2)
    out_desc(cc).start()
    if cc + 2 < _NCHUNK:
      gather_desc(cc + 2).start()
  out_desc(_NCHUNK - 2).wait()
  out_desc(_NCHUNK - 1).wait()

  nch = _HC // _L

  for h in range(2):
    b0 = base + h * _HC
    if h == 0:
      slab0.wait()
    else:
      pltpu.async_copy(outtr_hbm.at[:, pl.ds(b0, _HC)], slab_v, sem_s).wait()

    def nacc(dd, accs):
      return tuple(accs[c] + jnp.abs(slab_v[dd, pl.ds(c * _L, _L)])
                   for c in range(nch))
    norms = lax.fori_loop(
        0, _R, nacc, tuple(jnp.zeros((_L,), jnp.float32) for _ in range(nch)))
    scales = [_scale_vec(n) for n in norms]

    @pl.loop(0, _R)
    def _scale_rows(dd):
      for c in range(nch):
        slab_v[dd, pl.ds(c * _L, _L)] = (
            slab_v[dd, pl.ds(c * _L, _L)] * scales[c])

    pltpu.async_copy(slab_v, outt_hbm.at[:, pl.ds(b0, _HC)], sem_s).wait()


@jax.jit
def _run(inds, W_region, W_kernel):
  n = W_region.shape[0]
  wrt = W_region.T

  k1 = pl.kernel(
      _k1_body,
      out_type=jax.ShapeDtypeStruct((_R, _B), jnp.float32),
      mesh=_mesh(),
      scratch_types=[
          pltpu.VMEM((_B,), jnp.int32),
          pltpu.VMEM((n,), jnp.float32),
          pltpu.VMEM((_IDXH,), jnp.float32),
          pltpu.SemaphoreType.DMA,
      ],
      compiler_params=_cparams(),
  )
  out_raw = k1(inds, wrt)

  k2 = pl.kernel(
      _k2_body,
      out_type=(
          jax.ShapeDtypeStruct((_R, _B), jnp.float32),
          jax.ShapeDtypeStruct((_B, _K), jnp.float32),
      ),
      mesh=_mesh(),
      scratch_types=(
          [pltpu.VMEM((_BW,), jnp.int32),
           pltpu.VMEM((_R, _HC), jnp.float32)]
          + [pltpu.VMEM((_CH, _K), jnp.float32)] * 4
          + [pltpu.SemaphoreType.DMA] * 5
      ),
      compiler_params=_cparams(),
  )
  out_t, out_k = k2(inds, W_kernel, out_raw)
  return out_t.T, out_k


def kernel(inds, W_region, W_kernel):
  return _run(inds.astype(jnp.int32), W_region, W_kernel)


# speedup vs baseline: 5.2614x; 1.3900x over previous
import dataclasses
import functools

import jax
import jax.numpy as jnp
from jax import lax
from jax.experimental import pallas as pl
from jax.experimental.pallas import tpu as pltpu
from jax.experimental.pallas import tpu_sc as plsc

_NC, _NS, _L = 2, 16, 16
_NW = _NC * _NS
_B = 16384
_R = 196
_K = 512
_BW = _B // _NW
_CH = 32
_NCHUNK = _BW // _CH
_KFULL = _K // _L
_IDXH = _B // 2
_DMAX = (_R + _NW - 1) // _NW
_COLS = _B // _NW
_HC = _COLS // 2


def _mesh():
  return plsc.VectorSubcoreMesh(core_axis_name="c", subcore_axis_name="s",
                                num_cores=_NC, num_subcores=_NS)


def _cparams():
  cp = pltpu.CompilerParams()
  if "needs_layout_passes" in pltpu.CompilerParams.__dataclass_fields__:
    cp = dataclasses.replace(cp, needs_layout_passes=False)
  return cp


def _tree_sum(vals):
  vals = list(vals)
  while len(vals) > 1:
    nxt = [a + b for a, b in zip(vals[0::2], vals[1::2])]
    if len(vals) % 2:
      nxt.append(vals[-1])
    vals = nxt
  return vals[0]


def _scale_vec(norm):
  return jnp.where(norm > 1.0, 1.0 / (norm + 1e-7),
                   jnp.ones((_L,), jnp.float32))


def _k1_body(idx_hbm, wrt_hbm, outt_hbm, idx_v, row_v, gout_v, sem):
  wid = lax.axis_index("s") * _NC + lax.axis_index("c")
  pltpu.sync_copy(idx_hbm, idx_v)

  @pl.loop(0, _DMAX)
  def _rows(j):
    d = wid + j * _NW

    @pl.when(d < _R)
    def _():
      with jax.named_scope("k1_dma_row"):
        pltpu.async_copy(wrt_hbm.at[d], row_v, sem).wait()
      for h in range(2):
        with jax.named_scope("k1_gather"):
          @plsc.parallel_loop(0, _IDXH // _L, unroll=8)
          def _gather(w):
            iv = idx_v[pl.ds(h * _IDXH + w * _L, _L)]
            gout_v[pl.ds(w * _L, _L)] = plsc.load_gather(row_v, [iv])
        with jax.named_scope("k1_out"):
          pltpu.sync_copy(gout_v, outt_hbm.at[d, pl.ds(h * _IDXH, _IDXH)])


def _k2_body(idx_hbm, wk_hbm, outtr_hbm, outt_hbm, outk_hbm,
             idx_v, slab_v, buf_k0, buf_k1, obuf_k0, obuf_k1,
             sem_s, gsem_k0, gsem_k1, osem_k0, osem_k1):
  wid = lax.axis_index("s") * _NC + lax.axis_index("c")
  base = wid * _BW
  pltpu.sync_copy(idx_hbm.at[pl.ds(base, _BW)], idx_v)

  bufs_k = (buf_k0, buf_k1)
  obufs_k = (obuf_k0, obuf_k1)
  gsems_k = (gsem_k0, gsem_k1)
  osems_k = (osem_k0, osem_k1)

  slab0 = pltpu.make_async_copy(
      outtr_hbm.at[:, pl.ds(base, _HC)], slab_v, sem_s)
  slab0.start()

  def gather_desc(cc):
    p = cc % 2
    s = idx_v.at[pl.ds(cc * _CH, _CH)]
    return pltpu.make_async_copy(wk_hbm.at[s], bufs_k[p], gsems_k[p])

  def out_desc(cc):
    p = cc % 2
    d = pl.ds(base + cc * _CH, _CH)
    return pltpu.make_async_copy(obufs_k[p], outk_hbm.at[d], osems_k[p])

  def compute(p):
    buf_k, obuf_k = bufs_k[p], obufs_k[p]

    @pl.loop(0, _CH)
    def _row_k(r):
      wins = [buf_k[r, pl.ds(j * _L, _L)] for j in range(_KFULL)]
      norm = jnp.broadcast_to(
          jnp.sum(_tree_sum([jnp.abs(w) for w in wins])), (_L,))
      scale = _scale_vec(norm)
      for j in range(_KFULL):
        obuf_k[r, pl.ds(j * _L, _L)] = wins[j] * scale

  gather_desc(0).start()
  gather_desc(1).start()
  for cc in range(_NCHUNK):
    gather_desc(cc).wait()
    if cc >= 2:
      out_desc(cc - 2).wait()
    compute(cc % 2)
    out_desc(cc).start()
    if cc + 2 < _NCHUNK:
      gather_desc(cc + 2).start()
  out_desc(_NCHUNK - 2).wait()
  out_desc(_NCHUNK - 1).wait()

  nch = _HC // _L

  for h in range(2):
    b0 = base + h * _HC
    if h == 0:
      slab0.wait()
    else:
      pltpu.async_copy(outtr_hbm.at[:, pl.ds(b0, _HC)], slab_v, sem_s).wait()

    def nacc(dd, accs):
      return tuple(accs[c] + jnp.abs(slab_v[dd, pl.ds(c * _L, _L)])
                   for c in range(nch))
    norms = lax.fori_loop(
        0, _R, nacc, tuple(jnp.zeros((_L,), jnp.float32) for _ in range(nch)))
    scales = [_scale_vec(n) for n in norms]

    @pl.loop(0, _R)
    def _scale_rows(dd):
      for c in range(nch):
        slab_v[dd, pl.ds(c * _L, _L)] = (
            slab_v[dd, pl.ds(c * _L, _L)] * scales[c])

    pltpu.async_copy(slab_v, outt_hbm.at[:, pl.ds(b0, _HC)], sem_s).wait()


@jax.jit
def _run(inds, W_region, W_kernel):
  n = W_region.shape[0]
  wrt = W_region.T

  k1 = pl.kernel(
      _k1_body,
      out_type=jax.ShapeDtypeStruct((_R, _B), jnp.float32),
      mesh=_mesh(),
      scratch_types=[
          pltpu.VMEM((_B,), jnp.int32),
          pltpu.VMEM((n,), jnp.float32),
          pltpu.VMEM((_IDXH,), jnp.float32),
          pltpu.SemaphoreType.DMA,
      ],
      compiler_params=_cparams(),
  )
  out_raw = k1(inds, wrt)

  k2 = pl.kernel(
      _k2_body,
      out_type=(
          jax.ShapeDtypeStruct((_R, _B), jnp.float32),
          jax.ShapeDtypeStruct((_B, _K), jnp.float32),
      ),
      mesh=_mesh(),
      scratch_types=(
          [pltpu.VMEM((_BW,), jnp.int32),
           pltpu.VMEM((_R, _HC), jnp.float32)]
          + [pltpu.VMEM((_CH, _K), jnp.float32)] * 4
          + [pltpu.SemaphoreType.DMA] * 5
      ),
      compiler_params=_cparams(),
  )
  out_t, out_k = k2(inds, W_kernel, out_raw)
  return out_t.T, out_k


def kernel(inds, W_region, W_kernel):
  return _run(inds.astype(jnp.int32), W_region, W_kernel)
